# baseline (speedup 1.0000x reference)
import functools

import jax
import jax.numpy as jnp
from jax import lax
from jax.experimental import pallas as pl
from jax.experimental.pallas import tpu as pltpu
from jax.experimental.pallas import tpu_sc as plsc

MAXN = 512
NC, NS = 2, 16
NW = NC * NS
CHUNK = 32


def _sc_pack(node_feature, node_num):
    total, d = node_feature.shape
    nb = node_num.shape[0]
    rows_out = nb * MAXN
    copy_per_w = total // NW
    zero_per_w = (rows_out - total) // NW
    ncc = copy_per_w // CHUNK
    nzc = zero_per_w // CHUNK
    assert copy_per_w * NW == total and ncc * CHUNK == copy_per_w
    assert zero_per_w * NW == rows_out - total and nzc * CHUNK == zero_per_w

    mesh = plsc.VectorSubcoreMesh(core_axis_name="c", subcore_axis_name="s")

    nslots = 4
    zchunk = 16
    nzd = zero_per_w // zchunk

    @functools.partial(
        pl.kernel,
        out_type=jax.ShapeDtypeStruct((rows_out, d), jnp.float32),
        mesh=mesh,
        scratch_types=[
            pltpu.VMEM((16,), jnp.int32),
            [pltpu.VMEM((CHUNK, d), jnp.float32) for _ in range(nslots)],
            pltpu.VMEM((zchunk, d), jnp.float32),
            [pltpu.SemaphoreType.DMA for _ in range(nslots)],
            pltpu.SemaphoreType.DMA,
            pltpu.SemaphoreType.DMA,
        ],
    )
    def k(nf_hbm, nn_hbm, out_hbm, nn_v, bufs, zbuf, sems, zsem, nnsem):
        wid = lax.axis_index("s") * NC + lax.axis_index("c")

        def src(j):
            return pl.multiple_of(copy_per_w * wid + CHUNK * j, CHUNK)

        h_in = {}
        h_out = {}
        for j in range(min(nslots, ncc)):
            h_in[j] = pltpu.async_copy(
                nf_hbm.at[pl.ds(src(j), CHUNK)], bufs[j], sems[j]
            )
        h_nn = pltpu.async_copy(nn_hbm, nn_v, nnsem)

        zero16 = jnp.zeros((16,), jnp.float32)

        def zrow(i, c):
            def zcol(kk, cc):
                zbuf[i, pl.ds(kk * 16, 16)] = zero16
                return cc

            return lax.fori_loop(0, d // 16, zcol, c)

        lax.fori_loop(0, zchunk, zrow, 0)

        h_nn.wait()
        nnv = nn_v[...]
        nn_s = [nnv[i] for i in range(nb)]
        cu_s, acc = [], 0
        for i in range(nb):
            cu_s.append(acc)
            acc = acc + nn_s[i]
        pv_s = [MAXN * i - cu_s[i] for i in range(nb)]

        def copy_dst(j):
            s = copy_per_w * wid + CHUNK * j
            o = s
            for i in range(1, nb):
                o = jnp.where(cu_s[i] <= s, s + pv_s[i], o)
            return pl.multiple_of(o, CHUNK)

        def zero_dst(j):
            q = zero_per_w * wid + zchunk * j
            o = nn_s[0] + q
            for i in range(1, nb):
                o = jnp.where(pv_s[i] <= q, MAXN * i + nn_s[i] + (q - pv_s[i]), o)
            return pl.multiple_of(o, zchunk)

        h_zero = []
        for j in range(nzd):
            h_zero.append(
                pltpu.async_copy(zbuf, out_hbm.at[pl.ds(zero_dst(j), zchunk)], zsem)
            )

        for j in range(ncc):
            sl = j % nslots
            if j >= nslots:
                h_out[j - nslots].wait()
                h_in[j] = pltpu.async_copy(
                    nf_hbm.at[pl.ds(src(j), CHUNK)], bufs[sl], sems[sl]
                )
            h_in[j].wait()
            h_out[j] = pltpu.async_copy(
                bufs[sl], out_hbm.at[pl.ds(copy_dst(j), CHUNK)], sems[sl]
            )
        for j in range(max(0, ncc - nslots), ncc):
            h_out[j].wait()
        for h in h_zero:
            h.wait()

    return k(node_feature, node_num.astype(jnp.int32))


def kernel(node_feature, edge_index, edge_feature, node_num, edge_num):
    b = node_num.shape[0]
    d = node_feature.shape[-1]
    flat = _sc_pack(node_feature, node_num)
    padded_feature = flat.reshape(b, MAXN, d)
    token_pos = jnp.broadcast_to(
        jnp.arange(MAXN, dtype=jnp.int32)[None, :], (b, MAXN)
    )
    padded_node_mask = token_pos < node_num[:, None]
    padded_index = jnp.where(
        padded_node_mask[:, :, None],
        jnp.stack([token_pos, token_pos], axis=-1).astype(jnp.int32),
        0,
    )
    padding_mask = token_pos >= (node_num + edge_num)[:, None]
    return padded_index, padded_feature, padding_mask, padded_node_mask


# speedup vs baseline: 1.0027x; 1.0027x over previous
import functools

import jax
import jax.numpy as jnp
from jax import lax
from jax.experimental import pallas as pl
from jax.experimental.pallas import tpu as pltpu
from jax.experimental.pallas import tpu_sc as plsc

MAXN = 512
NC, NS = 2, 16
NW = NC * NS
CHUNK = 32


def _sc_pack(node_feature, node_num):
    total, d = node_feature.shape
    nb = node_num.shape[0]
    rows_out = nb * MAXN
    copy_per_w = total // NW
    zero_per_w = (rows_out - total) // NW
    ncc = copy_per_w // CHUNK
    nzc = zero_per_w // CHUNK
    assert copy_per_w * NW == total and ncc * CHUNK == copy_per_w
    assert zero_per_w * NW == rows_out - total and nzc * CHUNK == zero_per_w

    mesh = plsc.VectorSubcoreMesh(core_axis_name="c", subcore_axis_name="s")

    sizes = [8, 24] + [CHUNK] * ((copy_per_w - 32) // CHUNK)
    starts = [sum(sizes[:j]) for j in range(len(sizes))]
    assert sum(sizes) == copy_per_w
    zchunk = 8
    nzd = zero_per_w // zchunk

    @functools.partial(
        pl.kernel,
        out_type=jax.ShapeDtypeStruct((rows_out, d), jnp.float32),
        mesh=mesh,
        scratch_types=[
            pltpu.VMEM((16,), jnp.int32),
            [pltpu.VMEM((sz, d), jnp.float32) for sz in sizes],
            pltpu.VMEM((zchunk, d), jnp.float32),
            [pltpu.SemaphoreType.DMA for _ in sizes],
            pltpu.SemaphoreType.DMA,
            pltpu.SemaphoreType.DMA,
        ],
    )
    def k(nf_hbm, nn_hbm, out_hbm, nn_v, bufs, zbuf, sems, zsem, nnsem):
        wid = lax.axis_index("s") * NC + lax.axis_index("c")

        def src(j):
            return pl.multiple_of(copy_per_w * wid + starts[j], 8)

        h_in = {}
        h_out = {}
        for j in range(len(sizes)):
            h_in[j] = pltpu.async_copy(
                nf_hbm.at[pl.ds(src(j), sizes[j])], bufs[j], sems[j]
            )
        h_nn = pltpu.async_copy(nn_hbm, nn_v, nnsem)

        zero16 = jnp.zeros((16,), jnp.float32)

        def zrow(i, c):
            def zcol(kk, cc):
                zbuf[i, pl.ds(kk * 16, 16)] = zero16
                return cc

            return lax.fori_loop(0, d // 16, zcol, c)

        lax.fori_loop(0, zchunk, zrow, 0)

        h_nn.wait()
        nnv = nn_v[...]
        nn_s = [nnv[i] for i in range(nb)]
        cu_s, acc = [], 0
        for i in range(nb):
            cu_s.append(acc)
            acc = acc + nn_s[i]
        pv_s = [MAXN * i - cu_s[i] for i in range(nb)]

        def copy_dst(j):
            s = copy_per_w * wid + starts[j]
            o = s
            for i in range(1, nb):
                o = jnp.where(cu_s[i] <= s, s + pv_s[i], o)
            return pl.multiple_of(o, 8)

        def zero_dst(j):
            q = zero_per_w * wid + zchunk * j
            o = nn_s[0] + q
            for i in range(1, nb):
                o = jnp.where(pv_s[i] <= q, MAXN * i + nn_s[i] + (q - pv_s[i]), o)
            return pl.multiple_of(o, 8)

        h_zero = []
        for j in range(nzd):
            h_zero.append(
                pltpu.async_copy(zbuf, out_hbm.at[pl.ds(zero_dst(j), zchunk)], zsem)
            )

        for j in range(len(sizes)):
            h_in[j].wait()
            h_out[j] = pltpu.async_copy(
                bufs[j], out_hbm.at[pl.ds(copy_dst(j), sizes[j])], sems[j]
            )
        for j in range(len(sizes)):
            h_out[j].wait()
        for h in h_zero:
            h.wait()

    return k(node_feature, node_num.astype(jnp.int32))


def kernel(node_feature, edge_index, edge_feature, node_num, edge_num):
    b = node_num.shape[0]
    d = node_feature.shape[-1]
    flat = _sc_pack(node_feature, node_num)
    padded_feature = flat.reshape(b, MAXN, d)
    token_pos = jnp.broadcast_to(
        jnp.arange(MAXN, dtype=jnp.int32)[None, :], (b, MAXN)
    )
    padded_node_mask = token_pos < node_num[:, None]
    padded_index = jnp.where(
        padded_node_mask[:, :, None],
        jnp.stack([token_pos, token_pos], axis=-1).astype(jnp.int32),
        0,
    )
    padding_mask = token_pos >= (node_num + edge_num)[:, None]
    return padded_index, padded_feature, padding_mask, padded_node_mask
